# Initial kernel scaffold; baseline (speedup 1.0000x reference)
import jax
import jax.numpy as jnp
from jax.experimental import pallas as pl


def kernel(ctx_embed, query_experts, query_pos, pred_len_emb, latents, W_lat_q, W_ctx_k, W_ctx_v, W_lat_out, W_step_q, W_lat_k, W_lat_v, W_step_out, W_g1, b_g1, W_g2, b_g2, pred_len):
    raise NotImplementedError("write your pallas kernel here")


# speedup vs baseline: 1.3194x; 1.3194x over previous
import functools

import jax
import jax.numpy as jnp
import numpy as np
from jax import lax
from jax.experimental import pallas as pl
from jax.experimental.pallas import tpu as pltpu

_F32 = jnp.float32


def _dot(a, b, dims=None):
    if dims is None:
        dims = (((1,), (0,)), ((), ()))
    return lax.dot_general(a, b, dims, preferred_element_type=_F32)


def _softmax(x):
    m = jnp.max(x, axis=-1, keepdims=True)
    e = jnp.exp(x - m)
    return e / jnp.sum(e, axis=-1, keepdims=True)


def _mm_body(x_ref, w_ref, o_ref):
    o_ref[...] = _dot(x_ref[...], w_ref[...])


def _matmul(x, w, bn=512):
    M, K = x.shape
    N = w.shape[1]
    bn = min(bn, N)
    return pl.pallas_call(
        _mm_body,
        grid=(N // bn,),
        in_specs=[
            pl.BlockSpec((M, K), lambda i: (0, 0)),
            pl.BlockSpec((K, bn), lambda i: (0, i)),
        ],
        out_specs=pl.BlockSpec((M, bn), lambda i: (0, i)),
        out_shape=jax.ShapeDtypeStruct((M, N), _F32),
    )(x, w)


def _prep_body(qp_ref, lv_ref, wq_ref, q2_ref):
    sb = qp_ref[...] + lv_ref[...]
    q2_ref[...] = _dot(sb, wq_ref[...])


def _prep_q2(qp, lv, w_step_q, bn=512):
    P, D = qp.shape
    return pl.pallas_call(
        _prep_body,
        grid=(D // bn,),
        in_specs=[
            pl.BlockSpec((P, D), lambda i: (0, 0)),
            pl.BlockSpec((1, D), lambda i: (0, 0)),
            pl.BlockSpec((D, bn), lambda i: (0, i)),
        ],
        out_specs=pl.BlockSpec((P, bn), lambda i: (0, i)),
        out_shape=jax.ShapeDtypeStruct((P, D), _F32),
    )(qp, lv, w_step_q)


def _kv_body(q1_ref, ctx_ref, wk_ref, wv_ref, s_ref, v1_ref, scale):
    c = ctx_ref[0]
    k1t = _dot(c, wk_ref[...])
    s_ref[0] = _dot(q1_ref[...], k1t,
                    (((1,), (1,)), ((), ()))) / scale
    v1_ref[0] = _dot(c, wv_ref[...])


def _kv_scores(q1, ctx, wk, wv, tt=256):
    Lq, D = q1.shape
    B, T, _ = ctx.shape
    return pl.pallas_call(
        functools.partial(_kv_body, scale=np.float32(D ** 0.5)),
        grid=(B, T // tt),
        in_specs=[
            pl.BlockSpec((Lq, D), lambda b, t: (0, 0)),
            pl.BlockSpec((1, tt, D), lambda b, t: (b, t, 0)),
            pl.BlockSpec((D, D), lambda b, t: (0, 0)),
            pl.BlockSpec((D, D), lambda b, t: (0, 0)),
        ],
        out_specs=[
            pl.BlockSpec((1, Lq, tt), lambda b, t: (b, 0, t)),
            pl.BlockSpec((1, tt, D), lambda b, t: (b, t, 0)),
        ],
        out_shape=[
            jax.ShapeDtypeStruct((B, Lq, T), _F32),
            jax.ShapeDtypeStruct((B, T, D), _F32),
        ],
    )(q1, ctx, wk, wv)


def _att_body(s_ref, v1_ref, wo_ref, o_ref):
    a = _softmax(s_ref[0])
    o1 = _dot(a, v1_ref[0])
    o_ref[0] = _dot(o1, wo_ref[...])


def _att(scores, v1, w_lat_out):
    B, Lq, T = scores.shape
    D = v1.shape[-1]
    return pl.pallas_call(
        _att_body,
        grid=(B,),
        in_specs=[
            pl.BlockSpec((1, Lq, T), lambda b: (b, 0, 0)),
            pl.BlockSpec((1, T, D), lambda b: (b, 0, 0)),
            pl.BlockSpec((D, D), lambda b: (0, 0)),
        ],
        out_specs=pl.BlockSpec((1, Lq, D), lambda b: (b, 0, 0)),
        out_shape=jax.ShapeDtypeStruct((B, Lq, D), _F32),
    )(scores, v1, w_lat_out)


def _sc_body(q2_ref, k2_ref, v2_ref, wo_ref, o_ref, scale):
    s2 = _dot(q2_ref[...], k2_ref[0], (((1,), (1,)), ((), ()))) / scale
    a2 = _softmax(s2)
    t = _dot(a2, v2_ref[0])
    o_ref[0] = _dot(t, wo_ref[...])


def _step_ctx(q2, k2r, v2r, w_step_out):
    P, D = q2.shape
    B, Lq, _ = k2r.shape
    return pl.pallas_call(
        functools.partial(_sc_body, scale=np.float32(D ** 0.5)),
        grid=(B,),
        in_specs=[
            pl.BlockSpec((P, D), lambda b: (0, 0)),
            pl.BlockSpec((1, Lq, D), lambda b: (b, 0, 0)),
            pl.BlockSpec((1, Lq, D), lambda b: (b, 0, 0)),
            pl.BlockSpec((D, D), lambda b: (0, 0)),
        ],
        out_specs=pl.BlockSpec((1, P, D), lambda b: (b, 0, 0)),
        out_shape=jax.ShapeDtypeStruct((B, P, D), _F32),
    )(q2, k2r, v2r, w_step_out)


def _gate_body(qp_ref, lv_ref, sc_ref, wg_ref, b1_ref, h_ref):
    sb = qp_ref[...] + lv_ref[...]
    x = jnp.concatenate([sb, sc_ref[0]], axis=-1)
    pre = _dot(x, wg_ref[...]) + b1_ref[...]
    h_ref[0] = 0.5 * pre * (1.0 + lax.erf(pre / np.float32(np.sqrt(2.0))))


def _gate_h(qp, lv, sc, w_g1, b1, bn=512):
    P, D = qp.shape
    B = sc.shape[0]
    return pl.pallas_call(
        _gate_body,
        grid=(D // bn, B),
        in_specs=[
            pl.BlockSpec((P, D), lambda n, b: (0, 0)),
            pl.BlockSpec((1, D), lambda n, b: (0, 0)),
            pl.BlockSpec((1, P, D), lambda n, b: (b, 0, 0)),
            pl.BlockSpec((2 * D, bn), lambda n, b: (0, n)),
            pl.BlockSpec((1, bn), lambda n, b: (0, n)),
        ],
        out_specs=pl.BlockSpec((1, P, bn), lambda n, b: (b, 0, n)),
        out_shape=jax.ShapeDtypeStruct((B, P, D), _F32),
    )(qp, lv, sc, w_g1, b1)


def _route_body(h_ref, wg2_ref, b2_ref, qe_ref, o_ref):
    h = h_ref[...]
    lg = _dot(h, wg2_ref[...],
              (((2,), (0,)), ((), ()))) + b2_ref[...]
    E = lg.shape[-1]
    eio = lax.broadcasted_iota(jnp.int32, lg.shape, 2)
    v0 = jnp.max(lg, axis=-1, keepdims=True)
    i0 = jnp.min(jnp.where(lg == v0, eio, E), axis=-1, keepdims=True)
    lg2 = jnp.where(eio == i0, -jnp.inf, lg)
    v1 = jnp.max(lg2, axis=-1, keepdims=True)
    i1 = jnp.min(jnp.where(lg2 == v1, eio, E), axis=-1, keepdims=True)
    ex = jnp.exp(v1 - v0)
    w0 = 1.0 / (1.0 + ex)
    w1 = ex / (1.0 + ex)
    w = jnp.where(eio == i0, w0, jnp.where(eio == i1, w1, 0.0))
    acc = w[:, :, 0:1] * qe_ref[0][None]
    for e in range(1, E):
        acc = acc + w[:, :, e:e + 1] * qe_ref[e][None]
    o_ref[...] = acc


def _route(h, w_g2, b2, qe, pt=64):
    B, P, D = h.shape
    E = w_g2.shape[1]
    return pl.pallas_call(
        _route_body,
        grid=(P // pt,),
        in_specs=[
            pl.BlockSpec((B, pt, D), lambda p: (0, p, 0)),
            pl.BlockSpec((D, E), lambda p: (0, 0)),
            pl.BlockSpec((1, E), lambda p: (0, 0)),
            pl.BlockSpec((E, pt, D), lambda p: (0, p, 0)),
        ],
        out_specs=pl.BlockSpec((B, pt, D), lambda p: (0, p, 0)),
        out_shape=jax.ShapeDtypeStruct((B, P, D), _F32),
    )(h, w_g2, b2, qe)


def kernel(ctx_embed, query_experts, query_pos, pred_len_emb, latents,
           W_lat_q, W_ctx_k, W_ctx_v, W_lat_out, W_step_q, W_lat_k, W_lat_v,
           W_step_out, W_g1, b_g1, W_g2, b_g2, pred_len):
    B, T, D = ctx_embed.shape
    P = query_pos.shape[0]
    Lq = latents.shape[0]

    lv = pred_len_emb[pred_len][None]
    b1 = b_g1[None]
    b2 = b_g2[None]

    q1 = _matmul(latents, W_lat_q)
    scores, v1 = _kv_scores(q1, ctx_embed, W_ctx_k, W_ctx_v)
    lat_ctx = _att(scores, v1, W_lat_out)

    lcf = lat_ctx.reshape(B * Lq, D)
    k2 = _matmul(lcf, W_lat_k).reshape(B, Lq, D)
    v2 = _matmul(lcf, W_lat_v).reshape(B, Lq, D)

    q2 = _prep_q2(query_pos, lv, W_step_q)
    sc = _step_ctx(q2, k2, v2, W_step_out)
    h = _gate_h(query_pos, lv, sc, W_g1, b1)

    return _route(h, W_g2, b2, query_experts[:, :P, :])
